# baseline (speedup 1.0000x reference)
import jax
import jax.numpy as jnp
from jax.experimental import pallas as pl
from jax.experimental.pallas import tpu as pltpu

B = 32
T = 50
H = 1024
K = 64
D = 20
R = B * T
RC = 160
G1 = R // RC


def _fused_kernel(x1_ref, x2_ref, sk_ref, wm1_ref, bm1_ref, wm2_ref,
                  bm2_ref, sel1_ref, sel2_ref, expc_ref, kiota_ref,
                  wflat_ref, benc_ref, wih_ref, bg_ref, h0_ref, c0_ref,
                  whh_ref, wdec_ref, bdec_ref,
                  out_ref, hout_ref, cout_ref, xg_scr, h_scr, c_scr):
    s = pl.program_id(0)

    @pl.when(s < G1)
    def _phase_a():
        f1 = jnp.dot(x1_ref[...], wm1_ref[...],
                     preferred_element_type=jnp.float32) + bm1_ref[...]
        f2 = jnp.dot(x2_ref[...], wm2_ref[...],
                     preferred_element_type=jnp.float32) + bm2_ref[...]
        tiled = (jnp.dot(f1.astype(jnp.bfloat16), sel1_ref[...],
                         preferred_element_type=jnp.float32) +
                 jnp.dot(f2.astype(jnp.bfloat16), sel2_ref[...],
                         preferred_element_type=jnp.float32))
        sk = sk_ref[...]
        p = jnp.where(expc_ref[...] == sk, tiled, 0.0)
        onehot = (kiota_ref[...] == sk).astype(jnp.bfloat16)
        tmp = (jnp.dot(p.astype(jnp.bfloat16), wflat_ref[...],
                       preferred_element_type=jnp.float32) +
               jnp.dot(onehot, benc_ref[...],
                       preferred_element_type=jnp.float32))
        xg_scr[pl.ds(s * RC, RC), :] = jnp.dot(
            tmp.astype(jnp.bfloat16), wih_ref[...],
            preferred_element_type=jnp.float32) + bg_ref[...]

    @pl.when(s >= G1)
    def _phase_b():
        t = s - G1

        @pl.when(t == 0)
        def _():
            h_scr[...] = h0_ref[...]
            c_scr[...] = c0_ref[...]

        h = h_scr[...]
        c = c_scr[...]
        gates = (xg_scr[pl.ds(t * B, B), :] +
                 jnp.dot(h.astype(jnp.bfloat16), whh_ref[...],
                         preferred_element_type=jnp.float32))
        i_g = gates[:, 0 * H:1 * H]
        f_g = gates[:, 1 * H:2 * H]
        g_g = gates[:, 2 * H:3 * H]
        o_g = gates[:, 3 * H:4 * H]
        c_new = jax.nn.sigmoid(f_g) * c + jax.nn.sigmoid(i_g) * jnp.tanh(g_g)
        h_new = jax.nn.sigmoid(o_g) * jnp.tanh(c_new)
        h_scr[...] = h_new
        c_scr[...] = c_new
        out_ref[0] = jnp.dot(h_new.astype(jnp.bfloat16), wdec_ref[...],
                             preferred_element_type=jnp.float32) + bdec_ref[...]

        @pl.when(t == T - 1)
        def _():
            hout_ref[...] = h_new
            cout_ref[...] = c_new


@jax.jit
def kernel(input_1, input_2, h0, c0, routers_info, W_m1, b_m1, W_m2, b_m2,
           W_enc, b_enc, W_ih, W_hh, b_ih, b_hh, W_dec, b_dec):
    bf16 = jnp.bfloat16
    x1 = input_1.transpose(1, 0, 2).reshape(R, 2)
    x2 = input_2.transpose(1, 0, 2).reshape(R, 1)
    sk = routers_info.T.reshape(R, 1)
    w_flat = W_enc.reshape(K * D, H).astype(bf16)
    benc_b = b_enc.astype(bf16)
    wih_t = W_ih.astype(bf16).T
    whh_t = W_hh.astype(bf16).T
    wdec_t = W_dec.astype(bf16).T
    b_gates = (b_ih + b_hh).reshape(1, 4 * H)
    cols = jnp.arange(K * D, dtype=jnp.int32)
    expc = (cols // D).reshape(1, K * D)
    dmod = cols % D
    sel = (dmod[None, :] == jnp.arange(D, dtype=jnp.int32)[:, None])
    sel = sel.astype(bf16)
    sel1, sel2 = sel[:10], sel[10:]
    kiota = jnp.arange(K, dtype=jnp.int32).reshape(1, K)

    ca = lambda i: (jnp.minimum(i, G1 - 1), 0)
    cw = lambda i: (0, 0)
    cb = lambda i: (jnp.maximum(i - G1, 0), 0, 0)

    out3, h_t, c_t = pl.pallas_call(
        _fused_kernel,
        grid=(G1 + T,),
        in_specs=[
            pl.BlockSpec((RC, 2), ca),
            pl.BlockSpec((RC, 1), ca),
            pl.BlockSpec((RC, 1), ca),
            pl.BlockSpec((2, 10), cw),
            pl.BlockSpec((1, 10), cw),
            pl.BlockSpec((1, 10), cw),
            pl.BlockSpec((1, 10), cw),
            pl.BlockSpec((10, K * D), cw),
            pl.BlockSpec((10, K * D), cw),
            pl.BlockSpec((1, K * D), cw),
            pl.BlockSpec((1, K), cw),
            pl.BlockSpec((K * D, H), cw),
            pl.BlockSpec((K, H), cw),
            pl.BlockSpec((H, 4 * H), cw),
            pl.BlockSpec((1, 4 * H), cw),
            pl.BlockSpec((B, H), cw),
            pl.BlockSpec((B, H), cw),
            pl.BlockSpec((H, 4 * H), cw),
            pl.BlockSpec((H, K), cw),
            pl.BlockSpec((1, K), cw),
        ],
        out_specs=[
            pl.BlockSpec((1, B, K), cb),
            pl.BlockSpec((B, H), cw),
            pl.BlockSpec((B, H), cw),
        ],
        out_shape=[
            jax.ShapeDtypeStruct((T, B, K), jnp.float32),
            jax.ShapeDtypeStruct((B, H), jnp.float32),
            jax.ShapeDtypeStruct((B, H), jnp.float32),
        ],
        scratch_shapes=[
            pltpu.VMEM((R, 4 * H), jnp.float32),
            pltpu.VMEM((B, H), jnp.float32),
            pltpu.VMEM((B, H), jnp.float32),
        ],
    )(x1, x2, sk, W_m1.T, b_m1.reshape(1, 10), W_m2.T, b_m2.reshape(1, 10),
      sel1, sel2, expc, kiota, w_flat, benc_b, wih_t, b_gates,
      h0, c0, whh_t, wdec_t, b_dec.reshape(1, K))

    output = out3.transpose(1, 0, 2).reshape(B * T, K)
    return (output, h_t, c_t)


# speedup vs baseline: 1.0536x; 1.0536x over previous
import jax
import jax.numpy as jnp
from jax.experimental import pallas as pl
from jax.experimental.pallas import tpu as pltpu

B = 32
T = 50
H = 1024
K = 64
D = 20
R = B * T
RC = 400
G1 = R // RC


def _precompute_kernel(x1_ref, x2_ref, sk_ref, wm1_ref, bm1_ref, wm2_ref,
                       bm2_ref, sel1_ref, sel2_ref, expc_ref, kiota_ref,
                       wflat_ref, benc_ref, wih_ref, bg_ref, xg_ref):
    f1 = jnp.dot(x1_ref[...], wm1_ref[...],
                 preferred_element_type=jnp.float32) + bm1_ref[...]
    f2 = jnp.dot(x2_ref[...], wm2_ref[...],
                 preferred_element_type=jnp.float32) + bm2_ref[...]
    tiled = (jnp.dot(f1.astype(jnp.bfloat16), sel1_ref[...],
                     preferred_element_type=jnp.float32) +
             jnp.dot(f2.astype(jnp.bfloat16), sel2_ref[...],
                     preferred_element_type=jnp.float32))
    sk = sk_ref[...]
    p = jnp.where(expc_ref[...] == sk, tiled, 0.0)
    onehot = (kiota_ref[...] == sk).astype(jnp.bfloat16)
    tmp = (jnp.dot(p.astype(jnp.bfloat16), wflat_ref[...],
                   preferred_element_type=jnp.float32) +
           jnp.dot(onehot, benc_ref[...],
                   preferred_element_type=jnp.float32))
    xg_ref[...] = jnp.dot(tmp.astype(jnp.bfloat16), wih_ref[...],
                          preferred_element_type=jnp.float32) + bg_ref[...]


def _recurrent_kernel(xg_ref, h0_ref, c0_ref, whh_ref, wdec_ref, bdec_ref,
                      out_ref, hout_ref, cout_ref, h_scr, c_scr):
    t = pl.program_id(0)

    @pl.when(t == 0)
    def _():
        h_scr[...] = h0_ref[...]
        c_scr[...] = c0_ref[...]

    h = h_scr[...]
    c = c_scr[...]
    gates = xg_ref[0] + jnp.dot(h.astype(jnp.bfloat16), whh_ref[...],
                                preferred_element_type=jnp.float32)
    i_g = gates[:, 0 * H:1 * H]
    f_g = gates[:, 1 * H:2 * H]
    g_g = gates[:, 2 * H:3 * H]
    o_g = gates[:, 3 * H:4 * H]
    c_new = jax.nn.sigmoid(f_g) * c + jax.nn.sigmoid(i_g) * jnp.tanh(g_g)
    h_new = jax.nn.sigmoid(o_g) * jnp.tanh(c_new)
    h_scr[...] = h_new
    c_scr[...] = c_new
    out_ref[0] = jnp.dot(h_new.astype(jnp.bfloat16), wdec_ref[...],
                         preferred_element_type=jnp.float32) + bdec_ref[...]
    hout_ref[...] = h_new
    cout_ref[...] = c_new


@jax.jit
def kernel(input_1, input_2, h0, c0, routers_info, W_m1, b_m1, W_m2, b_m2,
           W_enc, b_enc, W_ih, W_hh, b_ih, b_hh, W_dec, b_dec):
    bf16 = jnp.bfloat16
    x1 = input_1.transpose(1, 0, 2).reshape(R, 2)
    x2 = input_2.transpose(1, 0, 2).reshape(R, 1)
    sk = routers_info.T.reshape(R, 1)
    w_flat = W_enc.reshape(K * D, H).astype(bf16)
    benc_b = b_enc.astype(bf16)
    wih_t = W_ih.astype(bf16).T
    whh_t = W_hh.astype(bf16).T
    wdec_t = W_dec.astype(bf16).T
    b_gates = (b_ih + b_hh).reshape(1, 4 * H)
    cols = jnp.arange(K * D, dtype=jnp.int32)
    expc = (cols // D).reshape(1, K * D)
    dmod = cols % D
    sel = (dmod[None, :] == jnp.arange(D, dtype=jnp.int32)[:, None])
    sel = sel.astype(bf16)
    sel1, sel2 = sel[:10], sel[10:]
    kiota = jnp.arange(K, dtype=jnp.int32).reshape(1, K)

    xg = pl.pallas_call(
        _precompute_kernel,
        grid=(G1,),
        in_specs=[
            pl.BlockSpec((RC, 2), lambda i: (i, 0)),
            pl.BlockSpec((RC, 1), lambda i: (i, 0)),
            pl.BlockSpec((RC, 1), lambda i: (i, 0)),
            pl.BlockSpec((2, 10), lambda i: (0, 0)),
            pl.BlockSpec((1, 10), lambda i: (0, 0)),
            pl.BlockSpec((1, 10), lambda i: (0, 0)),
            pl.BlockSpec((1, 10), lambda i: (0, 0)),
            pl.BlockSpec((10, K * D), lambda i: (0, 0)),
            pl.BlockSpec((10, K * D), lambda i: (0, 0)),
            pl.BlockSpec((1, K * D), lambda i: (0, 0)),
            pl.BlockSpec((1, K), lambda i: (0, 0)),
            pl.BlockSpec((K * D, H), lambda i: (0, 0)),
            pl.BlockSpec((K, H), lambda i: (0, 0)),
            pl.BlockSpec((H, 4 * H), lambda i: (0, 0)),
            pl.BlockSpec((1, 4 * H), lambda i: (0, 0)),
        ],
        out_specs=pl.BlockSpec((RC, 4 * H), lambda i: (i, 0)),
        out_shape=jax.ShapeDtypeStruct((R, 4 * H), jnp.float32),
    )(x1, x2, sk, W_m1.T, b_m1.reshape(1, 10), W_m2.T, b_m2.reshape(1, 10),
      sel1, sel2, expc, kiota, w_flat, benc_b, wih_t, b_gates)

    xg3 = xg.reshape(T, B, 4 * H)
    out3, h_t, c_t = pl.pallas_call(
        _recurrent_kernel,
        grid=(T,),
        in_specs=[
            pl.BlockSpec((1, B, 4 * H), lambda t: (t, 0, 0)),
            pl.BlockSpec((B, H), lambda t: (0, 0)),
            pl.BlockSpec((B, H), lambda t: (0, 0)),
            pl.BlockSpec((H, 4 * H), lambda t: (0, 0)),
            pl.BlockSpec((H, K), lambda t: (0, 0)),
            pl.BlockSpec((1, K), lambda t: (0, 0)),
        ],
        out_specs=[
            pl.BlockSpec((1, B, K), lambda t: (t, 0, 0)),
            pl.BlockSpec((B, H), lambda t: (0, 0)),
            pl.BlockSpec((B, H), lambda t: (0, 0)),
        ],
        out_shape=[
            jax.ShapeDtypeStruct((T, B, K), jnp.float32),
            jax.ShapeDtypeStruct((B, H), jnp.float32),
            jax.ShapeDtypeStruct((B, H), jnp.float32),
        ],
        scratch_shapes=[
            pltpu.VMEM((B, H), jnp.float32),
            pltpu.VMEM((B, H), jnp.float32),
        ],
    )(xg3, h0, c0, whh_t, wdec_t, b_dec.reshape(1, K))

    output = out3.transpose(1, 0, 2).reshape(B * T, K)
    return (output, h_t, c_t)


# speedup vs baseline: 1.1559x; 1.0971x over previous
import jax
import jax.numpy as jnp
from jax.experimental import pallas as pl
from jax.experimental.pallas import tpu as pltpu

B = 32
T = 50
H = 1024
K = 64
D = 20
R = B * T
RC = 400
G1 = R // RC
SPG = 2


def _precompute_kernel(x1_ref, x2_ref, sk_ref, wm1_ref, bm1_ref, wm2_ref,
                       bm2_ref, sel1_ref, sel2_ref, expc_ref, kiota_ref,
                       wflat_ref, benc_ref, wih_ref, bg_ref, whh_ref,
                       xg_ref, whht_ref):
    whht_ref[...] = whh_ref[...].astype(jnp.bfloat16).T
    f1 = jnp.dot(x1_ref[...], wm1_ref[...],
                 preferred_element_type=jnp.float32) + bm1_ref[...]
    f2 = jnp.dot(x2_ref[...], wm2_ref[...],
                 preferred_element_type=jnp.float32) + bm2_ref[...]
    tiled = (jnp.dot(f1.astype(jnp.bfloat16), sel1_ref[...],
                     preferred_element_type=jnp.float32) +
             jnp.dot(f2.astype(jnp.bfloat16), sel2_ref[...],
                     preferred_element_type=jnp.float32))
    sk = sk_ref[...]
    p = jnp.where(expc_ref[...] == sk, tiled, 0.0)
    onehot = (kiota_ref[...] == sk).astype(jnp.bfloat16)
    tmp = (jnp.dot(p.astype(jnp.bfloat16), wflat_ref[...],
                   preferred_element_type=jnp.float32) +
           jnp.dot(onehot, benc_ref[...],
                   preferred_element_type=jnp.float32))
    xg_ref[...] = jnp.dot(tmp.astype(jnp.bfloat16), wih_ref[...],
                          preferred_element_type=jnp.float32) + bg_ref[...]


def _recurrent_kernel(xg_ref, h0_ref, c0_ref, whh_ref, wdec_ref, bdec_ref,
                      out_ref, hout_ref, cout_ref, h_scr, c_scr):
    t = pl.program_id(0)

    @pl.when(t == 0)
    def _():
        h_scr[...] = h0_ref[...]
        c_scr[...] = c0_ref[...]

    h = h_scr[...]
    c = c_scr[...]
    for step in range(SPG):
        gates = xg_ref[step] + jnp.dot(h.astype(jnp.bfloat16), whh_ref[...],
                                       preferred_element_type=jnp.float32)
        i_g = gates[:, 0 * H:1 * H]
        f_g = gates[:, 1 * H:2 * H]
        g_g = gates[:, 2 * H:3 * H]
        o_g = gates[:, 3 * H:4 * H]
        c = jax.nn.sigmoid(f_g) * c + jax.nn.sigmoid(i_g) * jnp.tanh(g_g)
        h = jax.nn.sigmoid(o_g) * jnp.tanh(c)
        out_ref[step] = jnp.dot(h.astype(jnp.bfloat16), wdec_ref[...],
                                preferred_element_type=jnp.float32) + bdec_ref[...]
    h_scr[...] = h
    c_scr[...] = c
    hout_ref[...] = h
    cout_ref[...] = c


@jax.jit
def kernel(input_1, input_2, h0, c0, routers_info, W_m1, b_m1, W_m2, b_m2,
           W_enc, b_enc, W_ih, W_hh, b_ih, b_hh, W_dec, b_dec):
    bf16 = jnp.bfloat16
    x1 = input_1.transpose(1, 0, 2).reshape(R, 2)
    x2 = input_2.transpose(1, 0, 2).reshape(R, 1)
    sk = routers_info.T.reshape(R, 1)
    w_flat = W_enc.reshape(K * D, H).astype(bf16)
    benc_b = b_enc.astype(bf16)
    wih_t = W_ih.astype(bf16).T
    wdec_t = W_dec.astype(bf16).T
    b_gates = (b_ih + b_hh).reshape(1, 4 * H)
    cols = jnp.arange(K * D, dtype=jnp.int32)
    expc = (cols // D).reshape(1, K * D)
    dmod = cols % D
    sel = (dmod[None, :] == jnp.arange(D, dtype=jnp.int32)[:, None])
    sel = sel.astype(bf16)
    sel1, sel2 = sel[:10], sel[10:]
    kiota = jnp.arange(K, dtype=jnp.int32).reshape(1, K)

    xg, whh_t = pl.pallas_call(
        _precompute_kernel,
        grid=(G1,),
        in_specs=[
            pl.BlockSpec((RC, 2), lambda i: (i, 0)),
            pl.BlockSpec((RC, 1), lambda i: (i, 0)),
            pl.BlockSpec((RC, 1), lambda i: (i, 0)),
            pl.BlockSpec((2, 10), lambda i: (0, 0)),
            pl.BlockSpec((1, 10), lambda i: (0, 0)),
            pl.BlockSpec((1, 10), lambda i: (0, 0)),
            pl.BlockSpec((1, 10), lambda i: (0, 0)),
            pl.BlockSpec((10, K * D), lambda i: (0, 0)),
            pl.BlockSpec((10, K * D), lambda i: (0, 0)),
            pl.BlockSpec((1, K * D), lambda i: (0, 0)),
            pl.BlockSpec((1, K), lambda i: (0, 0)),
            pl.BlockSpec((K * D, H), lambda i: (0, 0)),
            pl.BlockSpec((K, H), lambda i: (0, 0)),
            pl.BlockSpec((H, 4 * H), lambda i: (0, 0)),
            pl.BlockSpec((1, 4 * H), lambda i: (0, 0)),
            pl.BlockSpec((H, H), lambda i: (i, 0)),
        ],
        out_specs=[
            pl.BlockSpec((RC, 4 * H), lambda i: (i, 0)),
            pl.BlockSpec((H, H), lambda i: (0, i)),
        ],
        out_shape=[
            jax.ShapeDtypeStruct((R, 4 * H), jnp.float32),
            jax.ShapeDtypeStruct((H, 4 * H), bf16),
        ],
    )(x1, x2, sk, W_m1.T, b_m1.reshape(1, 10), W_m2.T, b_m2.reshape(1, 10),
      sel1, sel2, expc, kiota, w_flat, benc_b, wih_t, b_gates, W_hh)

    xg3 = xg.reshape(T, B, 4 * H)
    out3, h_t, c_t = pl.pallas_call(
        _recurrent_kernel,
        grid=(T // SPG,),
        in_specs=[
            pl.BlockSpec((SPG, B, 4 * H), lambda t: (t, 0, 0)),
            pl.BlockSpec((B, H), lambda t: (0, 0)),
            pl.BlockSpec((B, H), lambda t: (0, 0)),
            pl.BlockSpec((H, 4 * H), lambda t: (0, 0)),
            pl.BlockSpec((H, K), lambda t: (0, 0)),
            pl.BlockSpec((1, K), lambda t: (0, 0)),
        ],
        out_specs=[
            pl.BlockSpec((SPG, B, K), lambda t: (t, 0, 0)),
            pl.BlockSpec((B, H), lambda t: (0, 0)),
            pl.BlockSpec((B, H), lambda t: (0, 0)),
        ],
        out_shape=[
            jax.ShapeDtypeStruct((T, B, K), jnp.float32),
            jax.ShapeDtypeStruct((B, H), jnp.float32),
            jax.ShapeDtypeStruct((B, H), jnp.float32),
        ],
        scratch_shapes=[
            pltpu.VMEM((B, H), jnp.float32),
            pltpu.VMEM((B, H), jnp.float32),
        ],
    )(xg3, h0, c0, whh_t, wdec_t, b_dec.reshape(1, K))

    output = out3.transpose(1, 0, 2).reshape(B * T, K)
    return (output, h_t, c_t)


# speedup vs baseline: 1.1682x; 1.0106x over previous
import jax
import jax.numpy as jnp
from jax.experimental import pallas as pl
from jax.experimental.pallas import tpu as pltpu

B = 32
T = 50
H = 1024
K = 64
D = 20
R = B * T
RC = 400
G1 = R // RC
SPG = 5


def _precompute_kernel(x1_ref, x2_ref, sk_ref, wm1_ref, bm1_ref, wm2_ref,
                       bm2_ref, sel1_ref, sel2_ref, expc_ref, kiota_ref,
                       wflat_ref, benc_ref, wih_ref, bg_ref, whh_ref,
                       xg_ref, whht_ref):
    whht_ref[...] = whh_ref[...].astype(jnp.bfloat16).T
    f1 = jnp.dot(x1_ref[...], wm1_ref[...],
                 preferred_element_type=jnp.float32) + bm1_ref[...]
    f2 = jnp.dot(x2_ref[...], wm2_ref[...],
                 preferred_element_type=jnp.float32) + bm2_ref[...]
    tiled = (jnp.dot(f1.astype(jnp.bfloat16), sel1_ref[...],
                     preferred_element_type=jnp.float32) +
             jnp.dot(f2.astype(jnp.bfloat16), sel2_ref[...],
                     preferred_element_type=jnp.float32))
    sk = sk_ref[...]
    p = jnp.where(expc_ref[...] == sk, tiled, 0.0)
    onehot = (kiota_ref[...] == sk).astype(jnp.bfloat16)
    tmp = (jnp.dot(p.astype(jnp.bfloat16), wflat_ref[...],
                   preferred_element_type=jnp.float32) +
           jnp.dot(onehot, benc_ref[...],
                   preferred_element_type=jnp.float32))
    xg_ref[...] = jnp.dot(tmp.astype(jnp.bfloat16), wih_ref[...],
                          preferred_element_type=jnp.float32) + bg_ref[...]


def _recurrent_kernel(xg_ref, h0_ref, c0_ref, whh_ref, wdec_ref, bdec_ref,
                      out_ref, hout_ref, cout_ref, h_scr, c_scr):
    t = pl.program_id(0)

    @pl.when(t == 0)
    def _():
        h_scr[...] = h0_ref[...]
        c_scr[...] = c0_ref[...]

    h = h_scr[...]
    c = c_scr[...]
    for step in range(SPG):
        gates = xg_ref[step] + jnp.dot(h.astype(jnp.bfloat16), whh_ref[...],
                                       preferred_element_type=jnp.float32)
        i_g = gates[:, 0 * H:1 * H]
        f_g = gates[:, 1 * H:2 * H]
        g_g = gates[:, 2 * H:3 * H]
        o_g = gates[:, 3 * H:4 * H]
        c = jax.nn.sigmoid(f_g) * c + jax.nn.sigmoid(i_g) * jnp.tanh(g_g)
        h = jax.nn.sigmoid(o_g) * jnp.tanh(c)
        out_ref[step] = jnp.dot(h.astype(jnp.bfloat16), wdec_ref[...],
                                preferred_element_type=jnp.float32) + bdec_ref[...]
    h_scr[...] = h
    c_scr[...] = c
    hout_ref[...] = h
    cout_ref[...] = c


@jax.jit
def kernel(input_1, input_2, h0, c0, routers_info, W_m1, b_m1, W_m2, b_m2,
           W_enc, b_enc, W_ih, W_hh, b_ih, b_hh, W_dec, b_dec):
    bf16 = jnp.bfloat16
    x1 = input_1.transpose(1, 0, 2).reshape(R, 2)
    x2 = input_2.transpose(1, 0, 2).reshape(R, 1)
    sk = routers_info.T.reshape(R, 1)
    w_flat = W_enc.reshape(K * D, H).astype(bf16)
    benc_b = b_enc.astype(bf16)
    wih_t = W_ih.astype(bf16).T
    wdec_t = W_dec.astype(bf16).T
    b_gates = (b_ih + b_hh).reshape(1, 4 * H)
    cols = jnp.arange(K * D, dtype=jnp.int32)
    expc = (cols // D).reshape(1, K * D)
    dmod = cols % D
    sel = (dmod[None, :] == jnp.arange(D, dtype=jnp.int32)[:, None])
    sel = sel.astype(bf16)
    sel1, sel2 = sel[:10], sel[10:]
    kiota = jnp.arange(K, dtype=jnp.int32).reshape(1, K)

    xg, whh_t = pl.pallas_call(
        _precompute_kernel,
        grid=(G1,),
        in_specs=[
            pl.BlockSpec((RC, 2), lambda i: (i, 0)),
            pl.BlockSpec((RC, 1), lambda i: (i, 0)),
            pl.BlockSpec((RC, 1), lambda i: (i, 0)),
            pl.BlockSpec((2, 10), lambda i: (0, 0)),
            pl.BlockSpec((1, 10), lambda i: (0, 0)),
            pl.BlockSpec((1, 10), lambda i: (0, 0)),
            pl.BlockSpec((1, 10), lambda i: (0, 0)),
            pl.BlockSpec((10, K * D), lambda i: (0, 0)),
            pl.BlockSpec((10, K * D), lambda i: (0, 0)),
            pl.BlockSpec((1, K * D), lambda i: (0, 0)),
            pl.BlockSpec((1, K), lambda i: (0, 0)),
            pl.BlockSpec((K * D, H), lambda i: (0, 0)),
            pl.BlockSpec((K, H), lambda i: (0, 0)),
            pl.BlockSpec((H, 4 * H), lambda i: (0, 0)),
            pl.BlockSpec((1, 4 * H), lambda i: (0, 0)),
            pl.BlockSpec((H, H), lambda i: (i, 0)),
        ],
        out_specs=[
            pl.BlockSpec((RC, 4 * H), lambda i: (i, 0)),
            pl.BlockSpec((H, H), lambda i: (0, i)),
        ],
        out_shape=[
            jax.ShapeDtypeStruct((R, 4 * H), jnp.float32),
            jax.ShapeDtypeStruct((H, 4 * H), bf16),
        ],
    )(x1, x2, sk, W_m1.T, b_m1.reshape(1, 10), W_m2.T, b_m2.reshape(1, 10),
      sel1, sel2, expc, kiota, w_flat, benc_b, wih_t, b_gates, W_hh)

    xg3 = xg.reshape(T, B, 4 * H)
    out3, h_t, c_t = pl.pallas_call(
        _recurrent_kernel,
        grid=(T // SPG,),
        in_specs=[
            pl.BlockSpec((SPG, B, 4 * H), lambda t: (t, 0, 0)),
            pl.BlockSpec((B, H), lambda t: (0, 0)),
            pl.BlockSpec((B, H), lambda t: (0, 0)),
            pl.BlockSpec((H, 4 * H), lambda t: (0, 0)),
            pl.BlockSpec((H, K), lambda t: (0, 0)),
            pl.BlockSpec((1, K), lambda t: (0, 0)),
        ],
        out_specs=[
            pl.BlockSpec((SPG, B, K), lambda t: (t, 0, 0)),
            pl.BlockSpec((B, H), lambda t: (0, 0)),
            pl.BlockSpec((B, H), lambda t: (0, 0)),
        ],
        out_shape=[
            jax.ShapeDtypeStruct((T, B, K), jnp.float32),
            jax.ShapeDtypeStruct((B, H), jnp.float32),
            jax.ShapeDtypeStruct((B, H), jnp.float32),
        ],
        scratch_shapes=[
            pltpu.VMEM((B, H), jnp.float32),
            pltpu.VMEM((B, H), jnp.float32),
        ],
    )(xg3, h0, c0, whh_t, wdec_t, b_dec.reshape(1, K))

    output = out3.transpose(1, 0, 2).reshape(B * T, K)
    return (output, h_t, c_t)


# speedup vs baseline: 1.1729x; 1.0041x over previous
import jax
import jax.numpy as jnp
from jax.experimental import pallas as pl
from jax.experimental.pallas import tpu as pltpu

B = 32
T = 50
H = 1024
K = 64
D = 20
R = B * T
RC = 400
G1 = R // RC
SPG = 5


def _precompute_kernel(x1_ref, x2_ref, sk_ref, wm1_ref, bm1_ref, wm2_ref,
                       bm2_ref, sel1_ref, sel2_ref, expc_ref, kiota_ref,
                       wflat_ref, benc_ref, wih_ref, bg_ref, whh_ref,
                       xg_ref, whht_ref, wiht_s, tmp_s):
    i = pl.program_id(0)
    q = pl.program_id(1)

    @pl.when(i == 0)
    def _():
        wiht_s[:, pl.ds(q * H, H)] = wih_ref[...].astype(jnp.bfloat16).T
        whht_ref[...] = whh_ref[...].astype(jnp.bfloat16).T

    @pl.when(q == 0)
    def _():
        f1 = jnp.dot(x1_ref[...], wm1_ref[...],
                     preferred_element_type=jnp.float32) + bm1_ref[...]
        f2 = jnp.dot(x2_ref[...], wm2_ref[...],
                     preferred_element_type=jnp.float32) + bm2_ref[...]
        tiled = (jnp.dot(f1.astype(jnp.bfloat16), sel1_ref[...],
                         preferred_element_type=jnp.float32) +
                 jnp.dot(f2.astype(jnp.bfloat16), sel2_ref[...],
                         preferred_element_type=jnp.float32))
        sk = sk_ref[...]
        p = jnp.where(expc_ref[...] == sk, tiled, 0.0)
        onehot = (kiota_ref[...] == sk).astype(jnp.bfloat16)
        tmp = (jnp.dot(p.astype(jnp.bfloat16), wflat_ref[...],
                       preferred_element_type=jnp.float32) +
               jnp.dot(onehot, benc_ref[...],
                       preferred_element_type=jnp.float32))
        tmp_s[...] = tmp.astype(jnp.bfloat16)

    xg_ref[...] = jnp.dot(tmp_s[...], wiht_s[:, pl.ds(q * H, H)],
                          preferred_element_type=jnp.float32) + bg_ref[...]


def _recurrent_kernel(xg_ref, h0_ref, c0_ref, whh_ref, wdec_ref, bdec_ref,
                      out_ref, hout_ref, cout_ref, h_scr, c_scr):
    t = pl.program_id(0)

    @pl.when(t == 0)
    def _():
        h_scr[...] = h0_ref[...]
        c_scr[...] = c0_ref[...]

    h = h_scr[...]
    c = c_scr[...]
    for step in range(SPG):
        gates = xg_ref[step] + jnp.dot(h.astype(jnp.bfloat16), whh_ref[...],
                                       preferred_element_type=jnp.float32)
        i_g = gates[:, 0 * H:1 * H]
        f_g = gates[:, 1 * H:2 * H]
        g_g = gates[:, 2 * H:3 * H]
        o_g = gates[:, 3 * H:4 * H]
        c = jax.nn.sigmoid(f_g) * c + jax.nn.sigmoid(i_g) * jnp.tanh(g_g)
        h = jax.nn.sigmoid(o_g) * jnp.tanh(c)
        out_ref[step] = jnp.dot(h.astype(jnp.bfloat16), wdec_ref[...],
                                preferred_element_type=jnp.float32) + bdec_ref[...]
    h_scr[...] = h
    c_scr[...] = c
    hout_ref[...] = h
    cout_ref[...] = c


@jax.jit
def kernel(input_1, input_2, h0, c0, routers_info, W_m1, b_m1, W_m2, b_m2,
           W_enc, b_enc, W_ih, W_hh, b_ih, b_hh, W_dec, b_dec):
    bf16 = jnp.bfloat16
    x1 = input_1.transpose(1, 0, 2).reshape(R, 2)
    x2 = input_2.transpose(1, 0, 2).reshape(R, 1)
    sk = routers_info.T.reshape(R, 1)
    w_flat = W_enc.reshape(K * D, H).astype(bf16)
    benc_b = b_enc.astype(bf16)
    wdec_t = W_dec.astype(bf16).T
    b_gates = (b_ih + b_hh).reshape(1, 4 * H)
    cols = jnp.arange(K * D, dtype=jnp.int32)
    expc = (cols // D).reshape(1, K * D)
    dmod = cols % D
    sel = (dmod[None, :] == jnp.arange(D, dtype=jnp.int32)[:, None])
    sel = sel.astype(bf16)
    sel1, sel2 = sel[:10], sel[10:]
    kiota = jnp.arange(K, dtype=jnp.int32).reshape(1, K)

    ci = lambda i, q: (i, 0)
    cc = lambda i, q: (0, 0)
    cq = lambda i, q: (jnp.where(i == 0, q, 3), 0)
    cqo = lambda i, q: (0, jnp.where(i == 0, q, 3))
    xg, whh_t = pl.pallas_call(
        _precompute_kernel,
        grid=(G1, 4),
        in_specs=[
            pl.BlockSpec((RC, 2), ci),
            pl.BlockSpec((RC, 1), ci),
            pl.BlockSpec((RC, 1), ci),
            pl.BlockSpec((2, 10), cc),
            pl.BlockSpec((1, 10), cc),
            pl.BlockSpec((1, 10), cc),
            pl.BlockSpec((1, 10), cc),
            pl.BlockSpec((10, K * D), cc),
            pl.BlockSpec((10, K * D), cc),
            pl.BlockSpec((1, K * D), cc),
            pl.BlockSpec((1, K), cc),
            pl.BlockSpec((K * D, H), cc),
            pl.BlockSpec((K, H), cc),
            pl.BlockSpec((H, H), cq),
            pl.BlockSpec((1, H), lambda i, q: (0, q)),
            pl.BlockSpec((H, H), cq),
        ],
        out_specs=[
            pl.BlockSpec((RC, H), lambda i, q: (i, q)),
            pl.BlockSpec((H, H), cqo),
        ],
        out_shape=[
            jax.ShapeDtypeStruct((R, 4 * H), jnp.float32),
            jax.ShapeDtypeStruct((H, 4 * H), bf16),
        ],
        scratch_shapes=[
            pltpu.VMEM((H, 4 * H), bf16),
            pltpu.VMEM((RC, H), bf16),
        ],
    )(x1, x2, sk, W_m1.T, b_m1.reshape(1, 10), W_m2.T, b_m2.reshape(1, 10),
      sel1, sel2, expc, kiota, w_flat, benc_b, W_ih, b_gates, W_hh)

    xg3 = xg.reshape(T, B, 4 * H)
    out3, h_t, c_t = pl.pallas_call(
        _recurrent_kernel,
        grid=(T // SPG,),
        in_specs=[
            pl.BlockSpec((SPG, B, 4 * H), lambda t: (t, 0, 0)),
            pl.BlockSpec((B, H), lambda t: (0, 0)),
            pl.BlockSpec((B, H), lambda t: (0, 0)),
            pl.BlockSpec((H, 4 * H), lambda t: (0, 0)),
            pl.BlockSpec((H, K), lambda t: (0, 0)),
            pl.BlockSpec((1, K), lambda t: (0, 0)),
        ],
        out_specs=[
            pl.BlockSpec((SPG, B, K), lambda t: (t, 0, 0)),
            pl.BlockSpec((B, H), lambda t: (0, 0)),
            pl.BlockSpec((B, H), lambda t: (0, 0)),
        ],
        out_shape=[
            jax.ShapeDtypeStruct((T, B, K), jnp.float32),
            jax.ShapeDtypeStruct((B, H), jnp.float32),
            jax.ShapeDtypeStruct((B, H), jnp.float32),
        ],
        scratch_shapes=[
            pltpu.VMEM((B, H), jnp.float32),
            pltpu.VMEM((B, H), jnp.float32),
        ],
    )(xg3, h0, c0, whh_t, wdec_t, b_dec.reshape(1, K))

    output = out3.transpose(1, 0, 2).reshape(B * T, K)
    return (output, h_t, c_t)


# speedup vs baseline: 1.2166x; 1.0373x over previous
import jax
import jax.numpy as jnp
from jax.experimental import pallas as pl
from jax.experimental.pallas import tpu as pltpu

B = 32
T = 50
H = 1024
K = 64
D = 20
R = B * T
RC = 200
G1 = R // RC
WQ = 4 * H // G1
SPG = 5


def _precompute_kernel(x1_ref, x2_ref, sk_ref, wm1_ref, bm1_ref, wm2_ref,
                       bm2_ref, sel1_ref, sel2_ref, expc_ref, kiota_ref,
                       wflat_ref, benc_ref, wih_ref, bg_ref, whh_ref,
                       xg_ref, whht_ref, wiht_s):
    i = pl.program_id(0)

    @pl.when(i == 0)
    def _():
        wiht_s[...] = wih_ref[...].astype(jnp.bfloat16).T

    whht_ref[...] = whh_ref[...].astype(jnp.bfloat16).T

    f1 = jnp.dot(x1_ref[...], wm1_ref[...],
                 preferred_element_type=jnp.float32) + bm1_ref[...]
    f2 = jnp.dot(x2_ref[...], wm2_ref[...],
                 preferred_element_type=jnp.float32) + bm2_ref[...]
    tiled = (jnp.dot(f1.astype(jnp.bfloat16), sel1_ref[...],
                     preferred_element_type=jnp.float32) +
             jnp.dot(f2.astype(jnp.bfloat16), sel2_ref[...],
                     preferred_element_type=jnp.float32))
    sk = sk_ref[...]
    p = jnp.where(expc_ref[...] == sk, tiled, 0.0)
    onehot = (kiota_ref[...] == sk).astype(jnp.bfloat16)
    tmp = (jnp.dot(p.astype(jnp.bfloat16), wflat_ref[...],
                   preferred_element_type=jnp.float32) +
           jnp.dot(onehot, benc_ref[...],
                   preferred_element_type=jnp.float32))
    xg_ref[...] = jnp.dot(tmp.astype(jnp.bfloat16), wiht_s[...],
                          preferred_element_type=jnp.float32) + bg_ref[...]


def _recurrent_kernel(xg_ref, h0_ref, c0_ref, whh_ref, wdec_ref, bdec_ref,
                      out_ref, hout_ref, cout_ref, h_scr, c_scr):
    t = pl.program_id(0)

    @pl.when(t == 0)
    def _():
        h_scr[...] = h0_ref[...]
        c_scr[...] = c0_ref[...]

    h = h_scr[...]
    c = c_scr[...]
    for step in range(SPG):
        gates = xg_ref[step] + jnp.dot(h.astype(jnp.bfloat16), whh_ref[...],
                                       preferred_element_type=jnp.float32)
        i_g = gates[:, 0 * H:1 * H]
        f_g = gates[:, 1 * H:2 * H]
        g_g = gates[:, 2 * H:3 * H]
        o_g = gates[:, 3 * H:4 * H]
        c = jax.nn.sigmoid(f_g) * c + jax.nn.sigmoid(i_g) * jnp.tanh(g_g)
        h = jax.nn.sigmoid(o_g) * jnp.tanh(c)
        out_ref[step] = jnp.dot(h.astype(jnp.bfloat16), wdec_ref[...],
                                preferred_element_type=jnp.float32) + bdec_ref[...]
    h_scr[...] = h
    c_scr[...] = c
    hout_ref[...] = h
    cout_ref[...] = c


@jax.jit
def kernel(input_1, input_2, h0, c0, routers_info, W_m1, b_m1, W_m2, b_m2,
           W_enc, b_enc, W_ih, W_hh, b_ih, b_hh, W_dec, b_dec):
    bf16 = jnp.bfloat16
    x1 = input_1.transpose(1, 0, 2).reshape(R, 2)
    x2 = input_2.transpose(1, 0, 2).reshape(R, 1)
    sk = routers_info.T.reshape(R, 1)
    w_flat = W_enc.reshape(K * D, H).astype(bf16)
    benc_b = b_enc.astype(bf16)
    wdec_t = W_dec.astype(bf16).T
    b_gates = (b_ih + b_hh).reshape(1, 4 * H)
    cols = jnp.arange(K * D, dtype=jnp.int32)
    expc = (cols // D).reshape(1, K * D)
    dmod = cols % D
    sel = (dmod[None, :] == jnp.arange(D, dtype=jnp.int32)[:, None])
    sel = sel.astype(bf16)
    sel1, sel2 = sel[:10], sel[10:]
    kiota = jnp.arange(K, dtype=jnp.int32).reshape(1, K)

    ci = lambda i: (i, 0)
    cc = lambda i: (0, 0)
    xg, whh_t = pl.pallas_call(
        _precompute_kernel,
        grid=(G1,),
        in_specs=[
            pl.BlockSpec((RC, 2), ci),
            pl.BlockSpec((RC, 1), ci),
            pl.BlockSpec((RC, 1), ci),
            pl.BlockSpec((2, 10), cc),
            pl.BlockSpec((1, 10), cc),
            pl.BlockSpec((1, 10), cc),
            pl.BlockSpec((1, 10), cc),
            pl.BlockSpec((10, K * D), cc),
            pl.BlockSpec((10, K * D), cc),
            pl.BlockSpec((1, K * D), cc),
            pl.BlockSpec((1, K), cc),
            pl.BlockSpec((K * D, H), cc),
            pl.BlockSpec((K, H), cc),
            pl.BlockSpec((4 * H, H), cc),
            pl.BlockSpec((1, 4 * H), cc),
            pl.BlockSpec((WQ, H), ci),
        ],
        out_specs=[
            pl.BlockSpec((RC, 4 * H), ci),
            pl.BlockSpec((H, WQ), lambda i: (0, i)),
        ],
        out_shape=[
            jax.ShapeDtypeStruct((R, 4 * H), jnp.float32),
            jax.ShapeDtypeStruct((H, 4 * H), bf16),
        ],
        scratch_shapes=[
            pltpu.VMEM((H, 4 * H), bf16),
        ],
    )(x1, x2, sk, W_m1.T, b_m1.reshape(1, 10), W_m2.T, b_m2.reshape(1, 10),
      sel1, sel2, expc, kiota, w_flat, benc_b, W_ih, b_gates, W_hh)

    xg3 = xg.reshape(T, B, 4 * H)
    out3, h_t, c_t = pl.pallas_call(
        _recurrent_kernel,
        grid=(T // SPG,),
        in_specs=[
            pl.BlockSpec((SPG, B, 4 * H), lambda t: (t, 0, 0)),
            pl.BlockSpec((B, H), lambda t: (0, 0)),
            pl.BlockSpec((B, H), lambda t: (0, 0)),
            pl.BlockSpec((H, 4 * H), lambda t: (0, 0)),
            pl.BlockSpec((H, K), lambda t: (0, 0)),
            pl.BlockSpec((1, K), lambda t: (0, 0)),
        ],
        out_specs=[
            pl.BlockSpec((SPG, B, K), lambda t: (t, 0, 0)),
            pl.BlockSpec((B, H), lambda t: (0, 0)),
            pl.BlockSpec((B, H), lambda t: (0, 0)),
        ],
        out_shape=[
            jax.ShapeDtypeStruct((T, B, K), jnp.float32),
            jax.ShapeDtypeStruct((B, H), jnp.float32),
            jax.ShapeDtypeStruct((B, H), jnp.float32),
        ],
        scratch_shapes=[
            pltpu.VMEM((B, H), jnp.float32),
            pltpu.VMEM((B, H), jnp.float32),
        ],
    )(xg3, h0, c0, whh_t, wdec_t, b_dec.reshape(1, K))

    output = out3.transpose(1, 0, 2).reshape(B * T, K)
    return (output, h_t, c_t)
